# baseline (speedup 1.0000x reference)
import functools

import jax
import jax.numpy as jnp
from jax import lax
from jax.experimental import pallas as pl
from jax.experimental.pallas import tpu as pltpu
from jax.experimental.pallas import tpu_sc as plsc

_CHUNK = 256
_NBUF = 4


@functools.lru_cache(maxsize=None)
def _make_gather(num_rows: int, dim: int, batch: int):
    info = plsc.get_sparse_core_info()
    nw = info.num_cores * info.num_subcores
    assert batch % (8 * nw) == 0
    b_per_w = batch // nw
    assert b_per_w % _CHUNK == 0
    n_chunks = b_per_w // _CHUNK
    assert n_chunks % _NBUF == 0
    mesh = plsc.VectorSubcoreMesh(core_axis_name="c", subcore_axis_name="s")

    @functools.partial(
        pl.kernel,
        mesh=mesh,
        compiler_params=pltpu.CompilerParams(use_tc_tiling_on_sc=False),
        out_type=jax.ShapeDtypeStruct((batch, dim), jnp.float32),
        scratch_types=[
            pltpu.VMEM((b_per_w,), jnp.int32),
        ]
        + [pltpu.VMEM((_CHUNK, dim), jnp.float32)] * _NBUF
        + [pltpu.SemaphoreType.DMA] * (2 * _NBUF),
    )
    def gather_kernel(table_hbm, idx_hbm, out_hbm, idx_v, r0, r1, r2, r3,
                      g0, g1, g2, g3, w0, w1, w2, w3):
        wid = lax.axis_index("s") * info.num_cores + lax.axis_index("c")
        base = pl.multiple_of(wid * b_per_w, 8)
        pltpu.sync_copy(idx_hbm.at[pl.ds(base, b_per_w)], idx_v)

        rows = (r0, r1, r2, r3)
        gsem = (g0, g1, g2, g3)
        wsem = (w0, w1, w2, w3)

        def start_gather(h, b):
            off = pl.multiple_of(h * _CHUNK, 8)
            pltpu.async_copy(table_hbm.at[idx_v.at[pl.ds(off, _CHUNK)]],
                             rows[b], gsem[b])

        def wait_gather(b):
            pltpu.make_async_copy(table_hbm.at[idx_v.at[pl.ds(0, _CHUNK)]],
                                  rows[b], gsem[b]).wait()

        def start_write(g, b):
            off = pl.multiple_of(base + g * _CHUNK, 8)
            pltpu.async_copy(rows[b], out_hbm.at[pl.ds(off, _CHUNK)], wsem[b])

        def wait_write(b):
            pltpu.make_async_copy(rows[b], out_hbm.at[pl.ds(0, _CHUNK)],
                                  wsem[b]).wait()

        start_gather(0, 0)
        start_gather(1, 1)

        def ring_body(p, carry):
            for b in range(_NBUF):
                g = p * _NBUF + b
                wait_gather(b)
                start_write(g, b)
                h = g + 2
                bh = (b + 2) % _NBUF

                @pl.when(jnp.logical_and(h < n_chunks, h >= _NBUF))
                def _():
                    wait_write(bh)

                @pl.when(h < n_chunks)
                def _():
                    start_gather(h, bh)
            return carry

        lax.fori_loop(0, n_chunks // _NBUF, ring_body, 0)

        for b in range(_NBUF):
            wait_write(b)

    return gather_kernel


def kernel(ids, weight):
    batch, seq = ids.shape
    num_rows, dim = weight.shape
    flat_ids = ids.reshape(-1).astype(jnp.int32)
    gather = _make_gather(num_rows, dim, flat_ids.shape[0])
    out = gather(weight, flat_ids)
    return out.reshape(batch, seq, dim)


# speedup vs baseline: 1.0020x; 1.0020x over previous
import functools

import jax
import jax.numpy as jnp
from jax import lax
from jax.experimental import pallas as pl
from jax.experimental.pallas import tpu as pltpu
from jax.experimental.pallas import tpu_sc as plsc

_CHUNK = 128
_NBUF = 8
_DEPTH = 4


@functools.lru_cache(maxsize=None)
def _make_gather(num_rows: int, dim: int, batch: int):
    info = plsc.get_sparse_core_info()
    nw = info.num_cores * info.num_subcores
    assert batch % (8 * nw) == 0
    b_per_w = batch // nw
    assert b_per_w % _CHUNK == 0
    n_chunks = b_per_w // _CHUNK
    assert n_chunks % _NBUF == 0 and n_chunks >= _NBUF
    mesh = plsc.VectorSubcoreMesh(core_axis_name="c", subcore_axis_name="s")

    @functools.partial(
        pl.kernel,
        mesh=mesh,
        compiler_params=pltpu.CompilerParams(use_tc_tiling_on_sc=False),
        out_type=jax.ShapeDtypeStruct((batch, dim), jnp.float32),
        scratch_types=[
            pltpu.VMEM((b_per_w,), jnp.int32),
        ]
        + [pltpu.VMEM((_CHUNK, dim), jnp.float32)] * _NBUF
        + [pltpu.SemaphoreType.DMA] * (2 * _NBUF),
    )
    def gather_kernel(table_hbm, idx_hbm, out_hbm, idx_v, *bufs_and_sems):
        rows = bufs_and_sems[:_NBUF]
        gsem = bufs_and_sems[_NBUF:2 * _NBUF]
        wsem = bufs_and_sems[2 * _NBUF:]
        wid = lax.axis_index("s") * info.num_cores + lax.axis_index("c")
        base = pl.multiple_of(wid * b_per_w, 8)
        pltpu.sync_copy(idx_hbm.at[pl.ds(base, b_per_w)], idx_v)

        def start_gather(h, b):
            off = pl.multiple_of(h * _CHUNK, 8)
            pltpu.async_copy(table_hbm.at[idx_v.at[pl.ds(off, _CHUNK)]],
                             rows[b], gsem[b])

        def wait_gather(b):
            pltpu.make_async_copy(table_hbm.at[idx_v.at[pl.ds(0, _CHUNK)]],
                                  rows[b], gsem[b]).wait()

        def start_write(g, b):
            off = pl.multiple_of(base + g * _CHUNK, 8)
            pltpu.async_copy(rows[b], out_hbm.at[pl.ds(off, _CHUNK)], wsem[b])

        def wait_write(b):
            pltpu.make_async_copy(rows[b], out_hbm.at[pl.ds(0, _CHUNK)],
                                  wsem[b]).wait()

        for b in range(_DEPTH):
            start_gather(b, b)

        def ring_body(p, carry):
            for b in range(_NBUF):
                g = p * _NBUF + b
                wait_gather(b)
                start_write(g, b)
                h = g + _DEPTH
                bh = (b + _DEPTH) % _NBUF

                @pl.when(jnp.logical_and(h < n_chunks, h >= _NBUF))
                def _():
                    wait_write(bh)

                @pl.when(h < n_chunks)
                def _():
                    start_gather(h, bh)
            return carry

        lax.fori_loop(0, n_chunks // _NBUF, ring_body, 0)

        for b in range(_NBUF):
            wait_write(b)

    return gather_kernel


def kernel(ids, weight):
    batch, seq = ids.shape
    num_rows, dim = weight.shape
    flat_ids = ids.reshape(-1).astype(jnp.int32)
    gather = _make_gather(num_rows, dim, flat_ids.shape[0])
    out = gather(weight, flat_ids)
    return out.reshape(batch, seq, dim)
